# Initial kernel scaffold; baseline (speedup 1.0000x reference)
import jax
import jax.numpy as jnp
from jax.experimental import pallas as pl


def kernel(x_transaction, x_merchant, x_category, ei_tm, ei_tc, ei_mt, ei_ct, params):
    raise NotImplementedError("write your pallas kernel here")


# speedup vs baseline: 3.7011x; 3.7011x over previous
import functools

import jax
import jax.numpy as jnp
from jax import lax
from jax.experimental import pallas as pl
from jax.experimental.pallas import tpu as pltpu
from jax.experimental.pallas import tpu_sc as plsc

N_TX, N_M, N_C = 50000, 5000, 400
D_IN, H, L = 64, 128, 3
LSTM_H = (L * H) // 2

NC, NS = 2, 16
NW = NC * NS
CH = 112
K = 14
ROWS_W = CH * K
N_TXP = NW * ROWS_W
N_MP = 5008
N_CP = 408
SEGM = 5120
SEGC = 512
BT = 512
NBLK = N_TXP // BT

@functools.cache
def _sc_mesh():
    return plsc.VectorSubcoreMesh(core_axis_name="c", subcore_axis_name="s",
                                  num_cores=NC, num_subcores=NS)


def _sc_counts_body(midx_h, cidx_h, ones_h, zer_h, cntm_o, cntc_o,
                    midx, cidx, bufone, segm_s, segc_s):
    c = lax.axis_index("c")
    s = lax.axis_index("s")
    wid = s * NC + c
    pltpu.sync_copy(zer_h, segm_s.at[pl.ds(s * 320, 320)])
    pltpu.sync_copy(zer_h.at[pl.ds(0, 32)], segc_s.at[pl.ds(s * 32, 32)])
    pltpu.sync_copy(ones_h, bufone)
    pltpu.sync_copy(midx_h.at[wid], midx)
    pltpu.sync_copy(cidx_h.at[wid], cidx)
    plsc.subcore_barrier()

    def chunk(j, carry):
        pltpu.sync_copy(bufone, segm_s.at[midx.at[j]], add=True)
        pltpu.sync_copy(bufone, segc_s.at[cidx.at[j]], add=True)
        return carry

    lax.fori_loop(0, K, chunk, 0)
    plsc.subcore_barrier()
    pltpu.sync_copy(segm_s.at[pl.ds(s * 320, 320)], cntm_o.at[c, pl.ds(s * 320, 320)])
    pltpu.sync_copy(segc_s.at[pl.ds(s * 32, 32)], cntc_o.at[c, pl.ds(s * 32, 32)])


def _sc_layer_body(xt_h, zm_h, zc_h, midx_h, cidx_h, zer_h,
                   segm_o, segc_o, g_o,
                   midx, cidx, bufx, bufa, bufb, segm_s, segc_s):
    c = lax.axis_index("c")
    s = lax.axis_index("s")
    wid = s * NC + c
    pltpu.sync_copy(zer_h, segm_s.at[pl.ds(s * 320, 320)])
    pltpu.sync_copy(zer_h.at[pl.ds(0, 32)], segc_s.at[pl.ds(s * 32, 32)])
    pltpu.sync_copy(midx_h.at[wid], midx)
    pltpu.sync_copy(cidx_h.at[wid], cidx)
    plsc.subcore_barrier()

    def chunk(j, carry):
        base = wid * ROWS_W + j * CH
        pltpu.sync_copy(xt_h.at[pl.ds(base, CH)], bufx)
        pltpu.sync_copy(bufx, segm_s.at[midx.at[j]], add=True)
        pltpu.sync_copy(bufx, segc_s.at[cidx.at[j]], add=True)
        pltpu.sync_copy(zm_h.at[midx.at[j]], bufa)
        pltpu.sync_copy(zc_h.at[cidx.at[j]], bufb)

        def addrow(r, cy):
            for kk in range(H // 16):
                sl = pl.ds(kk * 16, 16)
                bufa[r, sl] = bufa[r, sl] + bufb[r, sl]
            return cy

        lax.fori_loop(0, CH, addrow, 0)
        pltpu.sync_copy(bufa, g_o.at[pl.ds(base, CH)])
        return carry

    lax.fori_loop(0, K, chunk, 0)
    plsc.subcore_barrier()
    pltpu.sync_copy(segm_s.at[pl.ds(s * 320, 320)], segm_o.at[c, pl.ds(s * 320, 320)])
    pltpu.sync_copy(segc_s.at[pl.ds(s * 32, 32)], segc_o.at[c, pl.ds(s * 32, 32)])


def _sc_counts(midx, cidx, ones_h, zer_h):
    f = pl.kernel(
        _sc_counts_body,
        out_type=[jax.ShapeDtypeStruct((NC, SEGM, H), jnp.float32),
                  jax.ShapeDtypeStruct((NC, SEGC, H), jnp.float32)],
        mesh=_sc_mesh(),
        scratch_types=[
            pltpu.VMEM((K, CH), jnp.int32),
            pltpu.VMEM((K, CH), jnp.int32),
            pltpu.VMEM((CH, H), jnp.float32),
            pltpu.VMEM_SHARED((SEGM, H), jnp.float32),
            pltpu.VMEM_SHARED((SEGC, H), jnp.float32),
        ],
    )
    return f(midx, cidx, ones_h, zer_h)


def _sc_layer(xt, zm, zc, midx, cidx, zer_h):
    f = pl.kernel(
        _sc_layer_body,
        out_type=[jax.ShapeDtypeStruct((NC, SEGM, H), jnp.float32),
                  jax.ShapeDtypeStruct((NC, SEGC, H), jnp.float32),
                  jax.ShapeDtypeStruct((N_TXP, H), jnp.float32)],
        mesh=_sc_mesh(),
        scratch_types=[
            pltpu.VMEM((K, CH), jnp.int32),
            pltpu.VMEM((K, CH), jnp.int32),
            pltpu.VMEM((CH, H), jnp.float32),
            pltpu.VMEM((CH, H), jnp.float32),
            pltpu.VMEM((CH, H), jnp.float32),
            pltpu.VMEM_SHARED((SEGM, H), jnp.float32),
            pltpu.VMEM_SHARED((SEGC, H), jnp.float32),
        ],
    )
    return f(xt, zm, zc, midx, cidx, zer_h)


def _enc_body(x_ref, w_ref, b_ref, o_ref):
    o_ref[...] = jnp.maximum(
        jnp.dot(x_ref[...], w_ref[...], preferred_element_type=jnp.float32)
        + b_ref[...], 0.0)


def _enc_tx(x, w, b):
    return pl.pallas_call(
        _enc_body,
        grid=(NBLK,),
        in_specs=[pl.BlockSpec((BT, D_IN), lambda i: (i, 0)),
                  pl.BlockSpec((D_IN, H), lambda i: (0, 0)),
                  pl.BlockSpec((1, H), lambda i: (0, 0))],
        out_specs=pl.BlockSpec((BT, H), lambda i: (i, 0)),
        out_shape=jax.ShapeDtypeStruct((N_TXP, H), jnp.float32),
    )(x, w, b)


def _tpre_body(x_ref, g_ref, w_ref, h_ref, st_ref, acc_ref):
    i = pl.program_id(0)
    h = (jnp.dot(x_ref[...], w_ref[...], preferred_element_type=jnp.float32)
         + g_ref[...])
    h_ref[...] = h

    @pl.when(i == 0)
    def _():
        acc_ref[...] = jnp.zeros_like(acc_ref)

    ridx = i * BT + lax.broadcasted_iota(jnp.int32, (BT, 1), 0)
    m = (ridx < N_TX).astype(jnp.float32)
    hm = h * m
    acc_ref[0:1, :] += jnp.sum(hm, axis=0, keepdims=True)
    acc_ref[1:2, :] += jnp.sum(hm * h, axis=0, keepdims=True)

    @pl.when(i == NBLK - 1)
    def _():
        st_ref[...] = acc_ref[...]


def _tpre(x, g, w):
    return pl.pallas_call(
        _tpre_body,
        grid=(NBLK,),
        in_specs=[pl.BlockSpec((BT, H), lambda i: (i, 0)),
                  pl.BlockSpec((BT, H), lambda i: (i, 0)),
                  pl.BlockSpec((H, H), lambda i: (0, 0))],
        out_specs=[pl.BlockSpec((BT, H), lambda i: (i, 0)),
                   pl.BlockSpec((8, H), lambda i: (0, 0))],
        out_shape=[jax.ShapeDtypeStruct((N_TXP, H), jnp.float32),
                   jax.ShapeDtypeStruct((8, H), jnp.float32)],
        scratch_shapes=[pltpu.VMEM((8, H), jnp.float32)],
    )(x, g, w)


def _bn_from_stats(h, st_ref, gamma_ref, beta_ref, n):
    mu = st_ref[0:1, :] / n
    ex2 = st_ref[1:2, :] / n
    var = ex2 - mu * mu
    inv = lax.rsqrt(var + 1e-5)
    return (h - mu) * inv * gamma_ref[...] + beta_ref[...]


def _tapply_body_res(h_ref, st_ref, g_ref, b_ref, r_ref, o_ref):
    xn = _bn_from_stats(h_ref[...], st_ref, g_ref, b_ref, float(N_TX))
    o_ref[...] = jnp.maximum(xn + r_ref[...], 0.0)


def _tapply_body(h_ref, st_ref, g_ref, b_ref, o_ref):
    xn = _bn_from_stats(h_ref[...], st_ref, g_ref, b_ref, float(N_TX))
    o_ref[...] = jnp.maximum(xn, 0.0)


def _tapply(h, st, gamma, beta, res=None):
    specs = [pl.BlockSpec((BT, H), lambda i: (i, 0)),
             pl.BlockSpec((8, H), lambda i: (0, 0)),
             pl.BlockSpec((1, H), lambda i: (0, 0)),
             pl.BlockSpec((1, H), lambda i: (0, 0))]
    args = [h, st, gamma, beta]
    body = _tapply_body
    if res is not None:
        specs.append(pl.BlockSpec((BT, H), lambda i: (i, 0)))
        args.append(res)
        body = _tapply_body_res
    return pl.pallas_call(
        body,
        grid=(NBLK,),
        in_specs=specs,
        out_specs=pl.BlockSpec((BT, H), lambda i: (i, 0)),
        out_shape=jax.ShapeDtypeStruct((N_TXP, H), jnp.float32),
    )(*args)


def _mc_enc_body(xm_ref, xc_ref, wm_ref, bm_ref, wc_ref, bc_ref,
                 wzm_ref, bzm_ref, wzc_ref, bzc_ref,
                 xmo_ref, xco_ref, zmo_ref, zco_ref):
    xm = jnp.maximum(
        jnp.dot(xm_ref[...], wm_ref[...], preferred_element_type=jnp.float32)
        + bm_ref[...], 0.0)
    xc = jnp.maximum(
        jnp.dot(xc_ref[...], wc_ref[...], preferred_element_type=jnp.float32)
        + bc_ref[...], 0.0)
    xmo_ref[...] = xm
    xco_ref[...] = xc
    zmo_ref[...] = jnp.dot(xm, wzm_ref[...],
                           preferred_element_type=jnp.float32) + bzm_ref[...]
    zco_ref[...] = jnp.dot(xc, wzc_ref[...],
                           preferred_element_type=jnp.float32) + bzc_ref[...]


def _mc_enc(xm, xc, wm, bm, wc, bc, wzm, bzm, wzc, bzc):
    return pl.pallas_call(
        _mc_enc_body,
        out_shape=[jax.ShapeDtypeStruct((N_MP, H), jnp.float32),
                   jax.ShapeDtypeStruct((N_CP, H), jnp.float32),
                   jax.ShapeDtypeStruct((N_MP, H), jnp.float32),
                   jax.ShapeDtypeStruct((N_CP, H), jnp.float32)],
    )(xm, xc, wm, bm, wc, bc, wzm, bzm, wzc, bzc)


def _node_update(seg_ref, cnt_ref, x_ref, wl_ref, bl_ref, wr_ref,
                 gam_ref, bet_ref, n_valid, with_res):
    seg = seg_ref[0] + seg_ref[1]
    cnt = jnp.maximum(cnt_ref[0] + cnt_ref[1], 1.0)
    agg = seg / cnt
    h = (jnp.dot(agg, wl_ref[...], preferred_element_type=jnp.float32)
         + bl_ref[...]
         + jnp.dot(x_ref[...], wr_ref[...], preferred_element_type=jnp.float32))
    rows = h.shape[0]
    ridx = lax.broadcasted_iota(jnp.int32, (rows, 1), 0)
    m = (ridx < n_valid).astype(jnp.float32)
    hm = h * m
    n = float(n_valid)
    mu = jnp.sum(hm, axis=0, keepdims=True) / n
    ex2 = jnp.sum(hm * h, axis=0, keepdims=True) / n
    var = ex2 - mu * mu
    xn = (h - mu) * lax.rsqrt(var + 1e-5) * gam_ref[...] + bet_ref[...]
    if with_res:
        xn = xn + x_ref[...]
    return jnp.maximum(xn, 0.0)


def _mc_layer_body(segm_ref, cntm_ref, xm_ref, segc_ref, cntc_ref, xc_ref,
                   wlm_ref, blm_ref, wrm_ref, gm_ref, bm_ref,
                   wlc_ref, blc_ref, wrc_ref, gc_ref, bc_ref,
                   wzm_ref, bzm_ref, wzc_ref, bzc_ref,
                   xmo_ref, xco_ref, zmo_ref, zco_ref,
                   *, with_res, with_z):
    xm = _node_update(segm_ref, cntm_ref, xm_ref, wlm_ref, blm_ref, wrm_ref,
                      gm_ref, bm_ref, N_M, with_res)
    xc = _node_update(segc_ref, cntc_ref, xc_ref, wlc_ref, blc_ref, wrc_ref,
                      gc_ref, bc_ref, N_C, with_res)
    xmo_ref[...] = xm
    xco_ref[...] = xc
    if with_z:
        zmo_ref[...] = jnp.dot(xm, wzm_ref[...],
                               preferred_element_type=jnp.float32) + bzm_ref[...]
        zco_ref[...] = jnp.dot(xc, wzc_ref[...],
                               preferred_element_type=jnp.float32) + bzc_ref[...]
    else:
        zmo_ref[...] = jnp.zeros_like(zmo_ref)
        zco_ref[...] = jnp.zeros_like(zco_ref)


def _mc_layer(segm, cntm, xm, segc, cntc, xc, weights, with_res, with_z):
    body = functools.partial(_mc_layer_body, with_res=with_res, with_z=with_z)
    return pl.pallas_call(
        body,
        out_shape=[jax.ShapeDtypeStruct((N_MP, H), jnp.float32),
                   jax.ShapeDtypeStruct((N_CP, H), jnp.float32),
                   jax.ShapeDtypeStruct((N_MP, H), jnp.float32),
                   jax.ShapeDtypeStruct((N_CP, H), jnp.float32)],
    )(segm, cntm, xm, segc, cntc, xc, *weights)


def _lstm_dir(xs, wih_ref, whh_ref, b_ref):
    h = None
    c = None
    hs = []
    for t in range(4):
        gates = []
        for gi in range(4):
            acc = jnp.dot(xs[t], wih_ref[gi],
                          preferred_element_type=jnp.float32) + b_ref[gi]
            if h is not None:
                acc = acc + jnp.dot(h, whh_ref[gi],
                                    preferred_element_type=jnp.float32)
            gates.append(acc)
        gi_, gf_, gg_, go_ = gates
        if c is None:
            c = jax.nn.sigmoid(gi_) * jnp.tanh(gg_)
        else:
            c = jax.nn.sigmoid(gf_) * c + jax.nn.sigmoid(gi_) * jnp.tanh(gg_)
        h = jax.nn.sigmoid(go_) * jnp.tanh(c)
        hs.append(h)
    return hs


def _final_body(x0_ref, x1_ref, x2_ref, x3_ref,
                wihf_ref, whhf_ref, bf_ref, wihb_ref, whhb_ref, bb_ref,
                waf_ref, wab_ref, wpre_ref, bpre_ref, wcls_ref, bcls_ref,
                o_ref):
    xs = [x0_ref[...], x1_ref[...], x2_ref[...], x3_ref[...]]
    fw = _lstm_dir(xs, wihf_ref, whhf_ref, bf_ref)
    bwr = _lstm_dir(xs[::-1], wihb_ref, whhb_ref, bb_ref)
    bw = bwr[::-1]
    a = [jnp.sum(fw[t] * waf_ref[...], axis=1, keepdims=True)
         + jnp.sum(bw[t] * wab_ref[...], axis=1, keepdims=True)
         for t in range(4)]
    amax = jnp.maximum(jnp.maximum(a[0], a[1]), jnp.maximum(a[2], a[3]))
    e = [jnp.exp(av - amax) for av in a]
    denom = e[0] + e[1] + e[2] + e[3]
    xt = sum((e[t] / denom) * xs[t] for t in range(4))
    pre = jnp.maximum(
        jnp.dot(xt, wpre_ref[...], preferred_element_type=jnp.float32)
        + bpre_ref[...], 0.0)
    o_ref[...] = (jnp.dot(pre, wcls_ref[...],
                          preferred_element_type=jnp.float32) + bcls_ref[...])


def _final(xts, lw):
    full2 = lambda shape: pl.BlockSpec(shape, lambda i: (0, 0))
    full3 = lambda shape: pl.BlockSpec(shape, lambda i: (0, 0, 0))
    blk = pl.BlockSpec((BT, H), lambda i: (i, 0))
    return pl.pallas_call(
        _final_body,
        grid=(NBLK,),
        in_specs=[blk, blk, blk, blk,
                  full3((4, H, LSTM_H)), full3((4, LSTM_H, LSTM_H)),
                  full3((4, 1, LSTM_H)),
                  full3((4, H, LSTM_H)), full3((4, LSTM_H, LSTM_H)),
                  full3((4, 1, LSTM_H)),
                  full2((1, LSTM_H)), full2((1, LSTM_H)),
                  full2((H, H)), full2((1, H)),
                  full2((H, N_C)), full2((1, N_C))],
        out_specs=pl.BlockSpec((BT, N_C), lambda i: (i, 0)),
        out_shape=jax.ShapeDtypeStruct((N_TXP, N_C), jnp.float32),
    )(*xts, *lw)


def kernel(x_transaction, x_merchant, x_category, ei_tm, ei_tc, ei_mt, ei_ct,
           params):
    f32 = jnp.float32
    merch = ei_tm[1].astype(jnp.int32)
    cat = ei_tc[1].astype(jnp.int32)

    x_tp = jnp.pad(x_transaction, ((0, N_TXP - N_TX), (0, 0)))
    x_mp = jnp.pad(x_merchant, ((0, N_MP - N_M), (0, 0)))
    x_cp = jnp.pad(x_category, ((0, N_CP - N_C), (0, 0)))
    midx = jnp.pad(merch, (0, N_TXP - N_TX),
                   constant_values=N_M).reshape(NW, K, CH)
    cidx = jnp.pad(cat, (0, N_TXP - N_TX),
                   constant_values=N_C).reshape(NW, K, CH)
    zer_h = jnp.zeros((320, H), f32)
    ones_h = jnp.ones((CH, H), f32)

    p = params
    row = lambda v: v.reshape(1, -1)

    cntm, cntc = _sc_counts(midx, cidx, ones_h, zer_h)
    cntm_s = cntm[:, :N_MP, :]
    cntc_s = cntc[:, :N_CP, :]

    x_t = _enc_tx(x_tp, p['enc']['transaction']['W'],
                  row(p['enc']['transaction']['b']))
    cv0 = p['convs'][0]
    xm, xc, zm, zc = _mc_enc(
        x_mp, x_cp,
        p['enc']['merchant']['W'], row(p['enc']['merchant']['b']),
        p['enc']['category']['W'], row(p['enc']['category']['b']),
        cv0['mt']['Wl'], row(cv0['mt']['bl']),
        cv0['ct']['Wl'], row(cv0['ct']['bl']))

    xts = [x_t]
    for i in range(L):
        cv = p['convs'][i]
        bn = p['bn'][i]
        segm, segc, g = _sc_layer(x_t, zm, zc, midx, cidx, zer_h)
        wr_sum = cv['mt']['Wr'] + cv['ct']['Wr']
        h_pre, st = _tpre(x_t, g, wr_sum)
        x_t_new = _tapply(h_pre, st,
                          row(bn['transaction']['gamma']),
                          row(bn['transaction']['beta']),
                          res=x_t if i > 0 else None)
        with_z = i < L - 1
        cvn = p['convs'][i + 1] if with_z else cv
        weights = [
            cv['tm']['Wl'], row(cv['tm']['bl']), cv['tm']['Wr'],
            row(bn['merchant']['gamma']), row(bn['merchant']['beta']),
            cv['tc']['Wl'], row(cv['tc']['bl']), cv['tc']['Wr'],
            row(bn['category']['gamma']), row(bn['category']['beta']),
            cvn['mt']['Wl'], row(cvn['mt']['bl']),
            cvn['ct']['Wl'], row(cvn['ct']['bl']),
        ]
        xm, xc, zm, zc = _mc_layer(segm[:, :N_MP, :], cntm_s, xm,
                                   segc[:, :N_CP, :], cntc_s, xc,
                                   weights, with_res=i > 0, with_z=with_z)
        x_t = x_t_new
        xts.append(x_t)

    def lstm_prep(lp):
        wih = lp['Wih'].reshape(4, LSTM_H, H).transpose(0, 2, 1)
        whh = lp['Whh'].reshape(4, LSTM_H, LSTM_H).transpose(0, 2, 1)
        b = (lp['bih'] + lp['bhh']).reshape(4, 1, LSTM_H)
        return wih, whh, b

    wihf, whhf, bf = lstm_prep(p['lstm']['fw'])
    wihb, whhb, bb = lstm_prep(p['lstm']['bw'])
    waf = p['att']['W'][:LSTM_H, 0].reshape(1, LSTM_H)
    wab = p['att']['W'][LSTM_H:, 0].reshape(1, LSTM_H)
    lw = [wihf, whhf, bf, wihb, whhb, bb, waf, wab,
          p['pre']['W'], row(p['pre']['b']),
          p['cls']['W'], row(p['cls']['b'])]
    out = _final(xts, lw)
    return out[:N_TX]


# speedup vs baseline: 4.6488x; 1.2561x over previous
import functools

import jax
import jax.numpy as jnp
from jax import lax
from jax.experimental import pallas as pl
from jax.experimental.pallas import tpu as pltpu
from jax.experimental.pallas import tpu_sc as plsc

N_TX, N_M, N_C = 50000, 5000, 400
D_IN, H, L = 64, 128, 3
LSTM_H = (L * H) // 2

NC, NS = 2, 16
NW = NC * NS
CH = 112
K = 14
ROWS_W = CH * K
N_TXP = NW * ROWS_W
N_MP = 5008
N_CP = 408
SEGM = 5120
SEGC = 512
BT = 512
NBLK = N_TXP // BT

@functools.cache
def _sc_mesh():
    return plsc.VectorSubcoreMesh(core_axis_name="c", subcore_axis_name="s",
                                  num_cores=NC, num_subcores=NS)


def _sc_counts_body(midx_h, cidx_h, ones_h, zer_h, cntm_o, cntc_o,
                    midx, cidx, bufone, segm_s, segc_s):
    c = lax.axis_index("c")
    s = lax.axis_index("s")
    wid = s * NC + c
    pltpu.sync_copy(zer_h, segm_s.at[pl.ds(s * 320, 320)])
    pltpu.sync_copy(zer_h.at[pl.ds(0, 32)], segc_s.at[pl.ds(s * 32, 32)])
    pltpu.sync_copy(ones_h, bufone)
    pltpu.sync_copy(midx_h.at[wid], midx)
    pltpu.sync_copy(cidx_h.at[wid], cidx)
    plsc.subcore_barrier()

    def chunk(j, carry):
        pltpu.sync_copy(bufone, segm_s.at[midx.at[j]], add=True)
        pltpu.sync_copy(bufone, segc_s.at[cidx.at[j]], add=True)
        return carry

    lax.fori_loop(0, K, chunk, 0)
    plsc.subcore_barrier()
    pltpu.sync_copy(segm_s.at[pl.ds(s * 320, 320)], cntm_o.at[c, pl.ds(s * 320, 320)])
    pltpu.sync_copy(segc_s.at[pl.ds(s * 32, 32)], cntc_o.at[c, pl.ds(s * 32, 32)])


def _sc_layer_body(xt_h, zm_h, zc_h, midx_h, cidx_h, zer_h,
                   segm_o, segc_o, g_o,
                   midx, cidx, bufx, bufa, bufb, segm_s, segc_s):
    c = lax.axis_index("c")
    s = lax.axis_index("s")
    wid = s * NC + c
    pltpu.sync_copy(zer_h, segm_s.at[pl.ds(s * 320, 320)])
    pltpu.sync_copy(zer_h.at[pl.ds(0, 32)], segc_s.at[pl.ds(s * 32, 32)])
    pltpu.sync_copy(midx_h.at[wid], midx)
    pltpu.sync_copy(cidx_h.at[wid], cidx)
    plsc.subcore_barrier()

    def chunk(j, carry):
        base = wid * ROWS_W + j * CH
        pltpu.sync_copy(xt_h.at[pl.ds(base, CH)], bufx)
        pltpu.sync_copy(bufx, segm_s.at[midx.at[j]], add=True)
        pltpu.sync_copy(bufx, segc_s.at[cidx.at[j]], add=True)
        pltpu.sync_copy(zm_h.at[midx.at[j]], bufa)
        pltpu.sync_copy(zc_h.at[cidx.at[j]], bufb)

        def addrow(r, cy):
            for kk in range(H // 16):
                sl = pl.ds(kk * 16, 16)
                bufa[r, sl] = bufa[r, sl] + bufb[r, sl]
            return cy

        lax.fori_loop(0, CH, addrow, 0)
        pltpu.sync_copy(bufa, g_o.at[pl.ds(base, CH)])
        return carry

    lax.fori_loop(0, K, chunk, 0)
    plsc.subcore_barrier()
    pltpu.sync_copy(segm_s.at[pl.ds(s * 320, 320)], segm_o.at[c, pl.ds(s * 320, 320)])
    pltpu.sync_copy(segc_s.at[pl.ds(s * 32, 32)], segc_o.at[c, pl.ds(s * 32, 32)])


def _sc_counts(midx, cidx, ones_h, zer_h):
    f = pl.kernel(
        _sc_counts_body,
        out_type=[jax.ShapeDtypeStruct((NC, SEGM, H), jnp.float32),
                  jax.ShapeDtypeStruct((NC, SEGC, H), jnp.float32)],
        mesh=_sc_mesh(),
        scratch_types=[
            pltpu.VMEM((K, CH), jnp.int32),
            pltpu.VMEM((K, CH), jnp.int32),
            pltpu.VMEM((CH, H), jnp.float32),
            pltpu.VMEM_SHARED((SEGM, H), jnp.float32),
            pltpu.VMEM_SHARED((SEGC, H), jnp.float32),
        ],
    )
    return f(midx, cidx, ones_h, zer_h)


def _sc_layer(xt, zm, zc, midx, cidx, zer_h):
    f = pl.kernel(
        _sc_layer_body,
        out_type=[jax.ShapeDtypeStruct((NC, SEGM, H), jnp.float32),
                  jax.ShapeDtypeStruct((NC, SEGC, H), jnp.float32),
                  jax.ShapeDtypeStruct((N_TXP, H), jnp.float32)],
        mesh=_sc_mesh(),
        scratch_types=[
            pltpu.VMEM((K, CH), jnp.int32),
            pltpu.VMEM((K, CH), jnp.int32),
            pltpu.VMEM((CH, H), jnp.float32),
            pltpu.VMEM((CH, H), jnp.float32),
            pltpu.VMEM((CH, H), jnp.float32),
            pltpu.VMEM_SHARED((SEGM, H), jnp.float32),
            pltpu.VMEM_SHARED((SEGC, H), jnp.float32),
        ],
    )
    return f(xt, zm, zc, midx, cidx, zer_h)


def _enc_body(x_ref, w_ref, b_ref, o_ref):
    o_ref[...] = jnp.maximum(
        jnp.dot(x_ref[...], w_ref[...], preferred_element_type=jnp.float32)
        + b_ref[...], 0.0)


def _enc_tx(x, w, b):
    return pl.pallas_call(
        _enc_body,
        grid=(NBLK,),
        in_specs=[pl.BlockSpec((BT, D_IN), lambda i: (i, 0)),
                  pl.BlockSpec((D_IN, H), lambda i: (0, 0)),
                  pl.BlockSpec((1, H), lambda i: (0, 0))],
        out_specs=pl.BlockSpec((BT, H), lambda i: (i, 0)),
        out_shape=jax.ShapeDtypeStruct((N_TXP, H), jnp.float32),
    )(x, w, b)


def _tpre_body(x_ref, g_ref, w_ref, h_ref, st_ref, acc_ref):
    i = pl.program_id(0)
    h = (jnp.dot(x_ref[...], w_ref[...], preferred_element_type=jnp.float32)
         + g_ref[...])
    h_ref[...] = h

    @pl.when(i == 0)
    def _():
        acc_ref[...] = jnp.zeros_like(acc_ref)

    ridx = i * BT + lax.broadcasted_iota(jnp.int32, (BT, 1), 0)
    m = (ridx < N_TX).astype(jnp.float32)
    hm = h * m
    acc_ref[0:1, :] += jnp.sum(hm, axis=0, keepdims=True)
    acc_ref[1:2, :] += jnp.sum(hm * h, axis=0, keepdims=True)

    @pl.when(i == NBLK - 1)
    def _():
        st_ref[...] = acc_ref[...]


def _tpre(x, g, w):
    return pl.pallas_call(
        _tpre_body,
        grid=(NBLK,),
        in_specs=[pl.BlockSpec((BT, H), lambda i: (i, 0)),
                  pl.BlockSpec((BT, H), lambda i: (i, 0)),
                  pl.BlockSpec((H, H), lambda i: (0, 0))],
        out_specs=[pl.BlockSpec((BT, H), lambda i: (i, 0)),
                   pl.BlockSpec((8, H), lambda i: (0, 0))],
        out_shape=[jax.ShapeDtypeStruct((N_TXP, H), jnp.float32),
                   jax.ShapeDtypeStruct((8, H), jnp.float32)],
        scratch_shapes=[pltpu.VMEM((8, H), jnp.float32)],
    )(x, g, w)


def _bn_from_stats(h, st_ref, gamma_ref, beta_ref, n):
    mu = st_ref[0:1, :] / n
    ex2 = st_ref[1:2, :] / n
    var = ex2 - mu * mu
    inv = lax.rsqrt(var + 1e-5)
    return (h - mu) * inv * gamma_ref[...] + beta_ref[...]


def _tapply_body_res(h_ref, st_ref, g_ref, b_ref, r_ref, o_ref):
    xn = _bn_from_stats(h_ref[...], st_ref, g_ref, b_ref, float(N_TX))
    o_ref[...] = jnp.maximum(xn + r_ref[...], 0.0)


def _tapply_body(h_ref, st_ref, g_ref, b_ref, o_ref):
    xn = _bn_from_stats(h_ref[...], st_ref, g_ref, b_ref, float(N_TX))
    o_ref[...] = jnp.maximum(xn, 0.0)


def _tapply(h, st, gamma, beta, res=None):
    specs = [pl.BlockSpec((BT, H), lambda i: (i, 0)),
             pl.BlockSpec((8, H), lambda i: (0, 0)),
             pl.BlockSpec((1, H), lambda i: (0, 0)),
             pl.BlockSpec((1, H), lambda i: (0, 0))]
    args = [h, st, gamma, beta]
    body = _tapply_body
    if res is not None:
        specs.append(pl.BlockSpec((BT, H), lambda i: (i, 0)))
        args.append(res)
        body = _tapply_body_res
    return pl.pallas_call(
        body,
        grid=(NBLK,),
        in_specs=specs,
        out_specs=pl.BlockSpec((BT, H), lambda i: (i, 0)),
        out_shape=jax.ShapeDtypeStruct((N_TXP, H), jnp.float32),
    )(*args)


def _mc_enc_body(xm_ref, xc_ref, wm_ref, bm_ref, wc_ref, bc_ref,
                 wzm_ref, bzm_ref, wzc_ref, bzc_ref,
                 xmo_ref, xco_ref, zmo_ref, zco_ref):
    xm = jnp.maximum(
        jnp.dot(xm_ref[...], wm_ref[...], preferred_element_type=jnp.float32)
        + bm_ref[...], 0.0)
    xc = jnp.maximum(
        jnp.dot(xc_ref[...], wc_ref[...], preferred_element_type=jnp.float32)
        + bc_ref[...], 0.0)
    xmo_ref[...] = xm
    xco_ref[...] = xc
    zmo_ref[...] = jnp.dot(xm, wzm_ref[...],
                           preferred_element_type=jnp.float32) + bzm_ref[...]
    zco_ref[...] = jnp.dot(xc, wzc_ref[...],
                           preferred_element_type=jnp.float32) + bzc_ref[...]


def _mc_enc(xm, xc, wm, bm, wc, bc, wzm, bzm, wzc, bzc):
    return pl.pallas_call(
        _mc_enc_body,
        out_shape=[jax.ShapeDtypeStruct((N_MP, H), jnp.float32),
                   jax.ShapeDtypeStruct((N_CP, H), jnp.float32),
                   jax.ShapeDtypeStruct((N_MP, H), jnp.float32),
                   jax.ShapeDtypeStruct((N_CP, H), jnp.float32)],
    )(xm, xc, wm, bm, wc, bc, wzm, bzm, wzc, bzc)


def _node_update(seg_ref, cnt_ref, x_ref, wl_ref, bl_ref, wr_ref,
                 gam_ref, bet_ref, n_valid, with_res):
    seg = seg_ref[0] + seg_ref[1]
    cnt = jnp.maximum(cnt_ref[0] + cnt_ref[1], 1.0)
    agg = seg / cnt
    h = (jnp.dot(agg, wl_ref[...], preferred_element_type=jnp.float32)
         + bl_ref[...]
         + jnp.dot(x_ref[...], wr_ref[...], preferred_element_type=jnp.float32))
    rows = h.shape[0]
    ridx = lax.broadcasted_iota(jnp.int32, (rows, 1), 0)
    m = (ridx < n_valid).astype(jnp.float32)
    hm = h * m
    n = float(n_valid)
    mu = jnp.sum(hm, axis=0, keepdims=True) / n
    ex2 = jnp.sum(hm * h, axis=0, keepdims=True) / n
    var = ex2 - mu * mu
    xn = (h - mu) * lax.rsqrt(var + 1e-5) * gam_ref[...] + bet_ref[...]
    if with_res:
        xn = xn + x_ref[...]
    return jnp.maximum(xn, 0.0)


def _mc_layer_body(segm_ref, cntm_ref, xm_ref, segc_ref, cntc_ref, xc_ref,
                   wlm_ref, blm_ref, wrm_ref, gm_ref, bm_ref,
                   wlc_ref, blc_ref, wrc_ref, gc_ref, bc_ref,
                   wzm_ref, bzm_ref, wzc_ref, bzc_ref,
                   xmo_ref, xco_ref, zmo_ref, zco_ref,
                   *, with_res, with_z):
    xm = _node_update(segm_ref, cntm_ref, xm_ref, wlm_ref, blm_ref, wrm_ref,
                      gm_ref, bm_ref, N_M, with_res)
    xc = _node_update(segc_ref, cntc_ref, xc_ref, wlc_ref, blc_ref, wrc_ref,
                      gc_ref, bc_ref, N_C, with_res)
    xmo_ref[...] = xm
    xco_ref[...] = xc
    if with_z:
        zmo_ref[...] = jnp.dot(xm, wzm_ref[...],
                               preferred_element_type=jnp.float32) + bzm_ref[...]
        zco_ref[...] = jnp.dot(xc, wzc_ref[...],
                               preferred_element_type=jnp.float32) + bzc_ref[...]
    else:
        zmo_ref[...] = jnp.zeros_like(zmo_ref)
        zco_ref[...] = jnp.zeros_like(zco_ref)


def _mc_layer(segm, cntm, xm, segc, cntc, xc, weights, with_res, with_z):
    body = functools.partial(_mc_layer_body, with_res=with_res, with_z=with_z)
    return pl.pallas_call(
        body,
        out_shape=[jax.ShapeDtypeStruct((N_MP, H), jnp.float32),
                   jax.ShapeDtypeStruct((N_CP, H), jnp.float32),
                   jax.ShapeDtypeStruct((N_MP, H), jnp.float32),
                   jax.ShapeDtypeStruct((N_CP, H), jnp.float32)],
    )(segm, cntm, xm, segc, cntc, xc, *weights)


def _lstm_dir(xs, wih_ref, whh_ref, b_ref):
    h = None
    c = None
    hs = []
    for t in range(4):
        gates = []
        for gi in range(4):
            acc = jnp.dot(xs[t], wih_ref[gi],
                          preferred_element_type=jnp.float32) + b_ref[gi]
            if h is not None:
                acc = acc + jnp.dot(h, whh_ref[gi],
                                    preferred_element_type=jnp.float32)
            gates.append(acc)
        gi_, gf_, gg_, go_ = gates
        if c is None:
            c = jax.nn.sigmoid(gi_) * jnp.tanh(gg_)
        else:
            c = jax.nn.sigmoid(gf_) * c + jax.nn.sigmoid(gi_) * jnp.tanh(gg_)
        h = jax.nn.sigmoid(go_) * jnp.tanh(c)
        hs.append(h)
    return hs


def _final_body(x0_ref, x1_ref, x2_ref, x3_ref,
                wihf_ref, whhf_ref, bf_ref, wihb_ref, whhb_ref, bb_ref,
                waf_ref, wab_ref, wpre_ref, bpre_ref, wcls_ref, bcls_ref,
                o_ref):
    xs = [x0_ref[...], x1_ref[...], x2_ref[...], x3_ref[...]]
    fw = _lstm_dir(xs, wihf_ref, whhf_ref, bf_ref)
    bwr = _lstm_dir(xs[::-1], wihb_ref, whhb_ref, bb_ref)
    bw = bwr[::-1]
    a = [jnp.sum(fw[t] * waf_ref[...], axis=1, keepdims=True)
         + jnp.sum(bw[t] * wab_ref[...], axis=1, keepdims=True)
         for t in range(4)]
    amax = jnp.maximum(jnp.maximum(a[0], a[1]), jnp.maximum(a[2], a[3]))
    e = [jnp.exp(av - amax) for av in a]
    denom = e[0] + e[1] + e[2] + e[3]
    xt = sum((e[t] / denom) * xs[t] for t in range(4))
    pre = jnp.maximum(
        jnp.dot(xt, wpre_ref[...], preferred_element_type=jnp.float32)
        + bpre_ref[...], 0.0)
    o_ref[...] = (jnp.dot(pre, wcls_ref[...],
                          preferred_element_type=jnp.float32) + bcls_ref[...])


def _final(xts, lw):
    full2 = lambda shape: pl.BlockSpec(shape, lambda i: (0, 0))
    full3 = lambda shape: pl.BlockSpec(shape, lambda i: (0, 0, 0))
    blk = pl.BlockSpec((BT, H), lambda i: (i, 0))
    return pl.pallas_call(
        _final_body,
        grid=(NBLK,),
        in_specs=[blk, blk, blk, blk,
                  full3((4, H, LSTM_H)), full3((4, LSTM_H, LSTM_H)),
                  full3((4, 1, LSTM_H)),
                  full3((4, H, LSTM_H)), full3((4, LSTM_H, LSTM_H)),
                  full3((4, 1, LSTM_H)),
                  full2((1, LSTM_H)), full2((1, LSTM_H)),
                  full2((H, H)), full2((1, H)),
                  full2((H, N_C)), full2((1, N_C))],
        out_specs=pl.BlockSpec((BT, N_C), lambda i: (i, 0)),
        out_shape=jax.ShapeDtypeStruct((N_TX, N_C), jnp.float32),
    )(*xts, *lw)


def kernel(x_transaction, x_merchant, x_category, ei_tm, ei_tc, ei_mt, ei_ct,
           params):
    f32 = jnp.float32
    merch = ei_tm[1].astype(jnp.int32)
    cat = ei_tc[1].astype(jnp.int32)

    x_tp = jnp.pad(x_transaction, ((0, N_TXP - N_TX), (0, 0)))
    x_mp = jnp.pad(x_merchant, ((0, N_MP - N_M), (0, 0)))
    x_cp = jnp.pad(x_category, ((0, N_CP - N_C), (0, 0)))
    midx = jnp.pad(merch, (0, N_TXP - N_TX),
                   constant_values=N_M).reshape(NW, K, CH)
    cidx = jnp.pad(cat, (0, N_TXP - N_TX),
                   constant_values=N_C).reshape(NW, K, CH)
    zer_h = jnp.zeros((320, H), f32)
    ones_h = jnp.ones((CH, H), f32)

    p = params
    row = lambda v: v.reshape(1, -1)

    cntm, cntc = _sc_counts(midx, cidx, ones_h, zer_h)
    cntm_s = cntm[:, :N_MP, :]
    cntc_s = cntc[:, :N_CP, :]

    x_t = _enc_tx(x_tp, p['enc']['transaction']['W'],
                  row(p['enc']['transaction']['b']))
    cv0 = p['convs'][0]
    xm, xc, zm, zc = _mc_enc(
        x_mp, x_cp,
        p['enc']['merchant']['W'], row(p['enc']['merchant']['b']),
        p['enc']['category']['W'], row(p['enc']['category']['b']),
        cv0['mt']['Wl'], row(cv0['mt']['bl']),
        cv0['ct']['Wl'], row(cv0['ct']['bl']))

    xts = [x_t]
    for i in range(L):
        cv = p['convs'][i]
        bn = p['bn'][i]
        segm, segc, g = _sc_layer(x_t, zm, zc, midx, cidx, zer_h)
        wr_sum = cv['mt']['Wr'] + cv['ct']['Wr']
        h_pre, st = _tpre(x_t, g, wr_sum)
        x_t_new = _tapply(h_pre, st,
                          row(bn['transaction']['gamma']),
                          row(bn['transaction']['beta']),
                          res=x_t if i > 0 else None)
        with_z = i < L - 1
        cvn = p['convs'][i + 1] if with_z else cv
        weights = [
            cv['tm']['Wl'], row(cv['tm']['bl']), cv['tm']['Wr'],
            row(bn['merchant']['gamma']), row(bn['merchant']['beta']),
            cv['tc']['Wl'], row(cv['tc']['bl']), cv['tc']['Wr'],
            row(bn['category']['gamma']), row(bn['category']['beta']),
            cvn['mt']['Wl'], row(cvn['mt']['bl']),
            cvn['ct']['Wl'], row(cvn['ct']['bl']),
        ]
        xm, xc, zm, zc = _mc_layer(segm[:, :N_MP, :], cntm_s, xm,
                                   segc[:, :N_CP, :], cntc_s, xc,
                                   weights, with_res=i > 0, with_z=with_z)
        x_t = x_t_new
        xts.append(x_t)

    def lstm_prep(lp):
        wih = lp['Wih'].reshape(4, LSTM_H, H).transpose(0, 2, 1)
        whh = lp['Whh'].reshape(4, LSTM_H, LSTM_H).transpose(0, 2, 1)
        b = (lp['bih'] + lp['bhh']).reshape(4, 1, LSTM_H)
        return wih, whh, b

    wihf, whhf, bf = lstm_prep(p['lstm']['fw'])
    wihb, whhb, bb = lstm_prep(p['lstm']['bw'])
    waf = p['att']['W'][:LSTM_H, 0].reshape(1, LSTM_H)
    wab = p['att']['W'][LSTM_H:, 0].reshape(1, LSTM_H)
    lw = [wihf, whhf, bf, wihb, whhb, bb, waf, wab,
          p['pre']['W'], row(p['pre']['b']),
          p['cls']['W'], row(p['cls']['b'])]
    return _final(xts, lw)


# speedup vs baseline: 4.8329x; 1.0396x over previous
import functools

import jax
import jax.numpy as jnp
from jax import lax
from jax.experimental import pallas as pl
from jax.experimental.pallas import tpu as pltpu
from jax.experimental.pallas import tpu_sc as plsc

N_TX, N_M, N_C = 50000, 5000, 400
D_IN, H, L = 64, 128, 3
LSTM_H = (L * H) // 2

NC, NS = 2, 16
NW = NC * NS
CH = 112
K = 14
ROWS_W = CH * K
N_TXP = NW * ROWS_W
N_MP = 5008
N_CP = 408
SEGM = 5120
SEGC = 512
BT = 512
NBLK = N_TXP // BT

@functools.cache
def _sc_mesh():
    return plsc.VectorSubcoreMesh(core_axis_name="c", subcore_axis_name="s",
                                  num_cores=NC, num_subcores=NS)


def _sc_counts_body(midx_h, cidx_h, ones_h, zer_h, cntm_o, cntc_o,
                    midx, cidx, bufone, segm_s, segc_s):
    c = lax.axis_index("c")
    s = lax.axis_index("s")
    wid = s * NC + c
    pltpu.sync_copy(zer_h, segm_s.at[pl.ds(s * 320, 320)])
    pltpu.sync_copy(zer_h.at[pl.ds(0, 32)], segc_s.at[pl.ds(s * 32, 32)])
    pltpu.sync_copy(ones_h, bufone)
    pltpu.sync_copy(midx_h.at[wid], midx)
    pltpu.sync_copy(cidx_h.at[wid], cidx)
    plsc.subcore_barrier()

    def chunk(j, carry):
        pltpu.sync_copy(bufone, segm_s.at[midx.at[j]], add=True)
        pltpu.sync_copy(bufone, segc_s.at[cidx.at[j]], add=True)
        return carry

    lax.fori_loop(0, K, chunk, 0)
    plsc.subcore_barrier()
    pltpu.sync_copy(segm_s.at[pl.ds(s * 320, 320)], cntm_o.at[c, pl.ds(s * 320, 320)])
    pltpu.sync_copy(segc_s.at[pl.ds(s * 32, 32)], cntc_o.at[c, pl.ds(s * 32, 32)])


def _sc_layer_body(xt_h, zm_h, zc_h, midx_h, cidx_h, zer_h,
                   segm_o, segc_o, g_o,
                   midx, cidx, bufx, bufa, bufb, segm_s, segc_s):
    c = lax.axis_index("c")
    s = lax.axis_index("s")
    wid = s * NC + c
    pltpu.sync_copy(zer_h, segm_s.at[pl.ds(s * 320, 320)])
    pltpu.sync_copy(zer_h.at[pl.ds(0, 32)], segc_s.at[pl.ds(s * 32, 32)])
    pltpu.sync_copy(midx_h.at[wid], midx)
    pltpu.sync_copy(cidx_h.at[wid], cidx)
    plsc.subcore_barrier()

    def chunk(j, carry):
        base = wid * ROWS_W + j * CH
        pltpu.sync_copy(xt_h.at[pl.ds(base, CH)], bufx)
        pltpu.sync_copy(bufx, segm_s.at[midx.at[j]], add=True)
        pltpu.sync_copy(bufx, segc_s.at[cidx.at[j]], add=True)
        pltpu.sync_copy(zm_h.at[midx.at[j]], bufa)
        pltpu.sync_copy(zc_h.at[cidx.at[j]], bufb)

        def addrow(r, cy):
            for kk in range(H // 16):
                sl = pl.ds(kk * 16, 16)
                bufa[r, sl] = bufa[r, sl] + bufb[r, sl]
            return cy

        lax.fori_loop(0, CH, addrow, 0)
        pltpu.sync_copy(bufa, g_o.at[pl.ds(base, CH)])
        return carry

    lax.fori_loop(0, K, chunk, 0)
    plsc.subcore_barrier()
    pltpu.sync_copy(segm_s.at[pl.ds(s * 320, 320)], segm_o.at[c, pl.ds(s * 320, 320)])
    pltpu.sync_copy(segc_s.at[pl.ds(s * 32, 32)], segc_o.at[c, pl.ds(s * 32, 32)])


def _sc_counts(midx, cidx, ones_h, zer_h):
    f = pl.kernel(
        _sc_counts_body,
        out_type=[jax.ShapeDtypeStruct((NC, SEGM, H), jnp.float32),
                  jax.ShapeDtypeStruct((NC, SEGC, H), jnp.float32)],
        mesh=_sc_mesh(),
        scratch_types=[
            pltpu.VMEM((K, CH), jnp.int32),
            pltpu.VMEM((K, CH), jnp.int32),
            pltpu.VMEM((CH, H), jnp.float32),
            pltpu.VMEM_SHARED((SEGM, H), jnp.float32),
            pltpu.VMEM_SHARED((SEGC, H), jnp.float32),
        ],
    )
    return f(midx, cidx, ones_h, zer_h)


def _sc_layer(xt, zm, zc, midx, cidx, zer_h):
    f = pl.kernel(
        _sc_layer_body,
        out_type=[jax.ShapeDtypeStruct((NC, SEGM, H), jnp.float32),
                  jax.ShapeDtypeStruct((NC, SEGC, H), jnp.float32),
                  jax.ShapeDtypeStruct((N_TXP, H), jnp.float32)],
        mesh=_sc_mesh(),
        scratch_types=[
            pltpu.VMEM((K, CH), jnp.int32),
            pltpu.VMEM((K, CH), jnp.int32),
            pltpu.VMEM((CH, H), jnp.float32),
            pltpu.VMEM((CH, H), jnp.float32),
            pltpu.VMEM((CH, H), jnp.float32),
            pltpu.VMEM_SHARED((SEGM, H), jnp.float32),
            pltpu.VMEM_SHARED((SEGC, H), jnp.float32),
        ],
    )
    return f(xt, zm, zc, midx, cidx, zer_h)


def _enc_body(x_ref, w_ref, b_ref, o_ref):
    o_ref[...] = jnp.maximum(
        jnp.dot(x_ref[...], w_ref[...], preferred_element_type=jnp.float32)
        + b_ref[...], 0.0)


def _enc_tx(x, w, b):
    return pl.pallas_call(
        _enc_body,
        grid=(NBLK,),
        in_specs=[pl.BlockSpec((BT, D_IN), lambda i: (i, 0)),
                  pl.BlockSpec((D_IN, H), lambda i: (0, 0)),
                  pl.BlockSpec((1, H), lambda i: (0, 0))],
        out_specs=pl.BlockSpec((BT, H), lambda i: (i, 0)),
        out_shape=jax.ShapeDtypeStruct((N_TXP, H), jnp.float32),
    )(x, w, b)


def _tpre_body(x_ref, g_ref, w_ref, h_ref, st_ref, acc_ref):
    i = pl.program_id(0)
    h = (jnp.dot(x_ref[...], w_ref[...], preferred_element_type=jnp.float32)
         + g_ref[...])
    h_ref[...] = h

    @pl.when(i == 0)
    def _():
        acc_ref[...] = jnp.zeros_like(acc_ref)

    ridx = i * BT + lax.broadcasted_iota(jnp.int32, (BT, 1), 0)
    m = (ridx < N_TX).astype(jnp.float32)
    hm = h * m
    acc_ref[0:1, :] += jnp.sum(hm, axis=0, keepdims=True)
    acc_ref[1:2, :] += jnp.sum(hm * h, axis=0, keepdims=True)

    @pl.when(i == NBLK - 1)
    def _():
        st_ref[...] = acc_ref[...]


def _tpre(x, g, w):
    return pl.pallas_call(
        _tpre_body,
        grid=(NBLK,),
        in_specs=[pl.BlockSpec((BT, H), lambda i: (i, 0)),
                  pl.BlockSpec((BT, H), lambda i: (i, 0)),
                  pl.BlockSpec((H, H), lambda i: (0, 0))],
        out_specs=[pl.BlockSpec((BT, H), lambda i: (i, 0)),
                   pl.BlockSpec((8, H), lambda i: (0, 0))],
        out_shape=[jax.ShapeDtypeStruct((N_TXP, H), jnp.float32),
                   jax.ShapeDtypeStruct((8, H), jnp.float32)],
        scratch_shapes=[pltpu.VMEM((8, H), jnp.float32)],
    )(x, g, w)


def _bn_from_stats(h, st_ref, gamma_ref, beta_ref, n):
    mu = st_ref[0:1, :] / n
    ex2 = st_ref[1:2, :] / n
    var = ex2 - mu * mu
    inv = lax.rsqrt(var + 1e-5)
    return (h - mu) * inv * gamma_ref[...] + beta_ref[...]


def _tapply_body_res(h_ref, st_ref, g_ref, b_ref, r_ref, o_ref):
    xn = _bn_from_stats(h_ref[...], st_ref, g_ref, b_ref, float(N_TX))
    o_ref[...] = jnp.maximum(xn + r_ref[...], 0.0)


def _tapply_body(h_ref, st_ref, g_ref, b_ref, o_ref):
    xn = _bn_from_stats(h_ref[...], st_ref, g_ref, b_ref, float(N_TX))
    o_ref[...] = jnp.maximum(xn, 0.0)


def _tapply(h, st, gamma, beta, res=None):
    specs = [pl.BlockSpec((BT, H), lambda i: (i, 0)),
             pl.BlockSpec((8, H), lambda i: (0, 0)),
             pl.BlockSpec((1, H), lambda i: (0, 0)),
             pl.BlockSpec((1, H), lambda i: (0, 0))]
    args = [h, st, gamma, beta]
    body = _tapply_body
    if res is not None:
        specs.append(pl.BlockSpec((BT, H), lambda i: (i, 0)))
        args.append(res)
        body = _tapply_body_res
    return pl.pallas_call(
        body,
        grid=(NBLK,),
        in_specs=specs,
        out_specs=pl.BlockSpec((BT, H), lambda i: (i, 0)),
        out_shape=jax.ShapeDtypeStruct((N_TXP, H), jnp.float32),
    )(*args)


def _mc_enc_body(xm_ref, xc_ref, wm_ref, bm_ref, wc_ref, bc_ref,
                 wzm_ref, bzm_ref, wzc_ref, bzc_ref,
                 xmo_ref, xco_ref, zmo_ref, zco_ref):
    xm = jnp.maximum(
        jnp.dot(xm_ref[...], wm_ref[...], preferred_element_type=jnp.float32)
        + bm_ref[...], 0.0)
    xc = jnp.maximum(
        jnp.dot(xc_ref[...], wc_ref[...], preferred_element_type=jnp.float32)
        + bc_ref[...], 0.0)
    xmo_ref[...] = xm
    xco_ref[...] = xc
    zmo_ref[...] = jnp.dot(xm, wzm_ref[...],
                           preferred_element_type=jnp.float32) + bzm_ref[...]
    zco_ref[...] = jnp.dot(xc, wzc_ref[...],
                           preferred_element_type=jnp.float32) + bzc_ref[...]


def _mc_enc(xm, xc, wm, bm, wc, bc, wzm, bzm, wzc, bzc):
    return pl.pallas_call(
        _mc_enc_body,
        out_shape=[jax.ShapeDtypeStruct((N_MP, H), jnp.float32),
                   jax.ShapeDtypeStruct((N_CP, H), jnp.float32),
                   jax.ShapeDtypeStruct((N_MP, H), jnp.float32),
                   jax.ShapeDtypeStruct((N_CP, H), jnp.float32)],
    )(xm, xc, wm, bm, wc, bc, wzm, bzm, wzc, bzc)


def _node_update(seg_ref, cnt_ref, x_ref, wl_ref, bl_ref, wr_ref,
                 gam_ref, bet_ref, n_valid, with_res):
    seg = seg_ref[0] + seg_ref[1]
    cnt = jnp.maximum(cnt_ref[0] + cnt_ref[1], 1.0)
    agg = seg / cnt
    h = (jnp.dot(agg, wl_ref[...], preferred_element_type=jnp.float32)
         + bl_ref[...]
         + jnp.dot(x_ref[...], wr_ref[...], preferred_element_type=jnp.float32))
    rows = h.shape[0]
    ridx = lax.broadcasted_iota(jnp.int32, (rows, 1), 0)
    m = (ridx < n_valid).astype(jnp.float32)
    hm = h * m
    n = float(n_valid)
    mu = jnp.sum(hm, axis=0, keepdims=True) / n
    ex2 = jnp.sum(hm * h, axis=0, keepdims=True) / n
    var = ex2 - mu * mu
    xn = (h - mu) * lax.rsqrt(var + 1e-5) * gam_ref[...] + bet_ref[...]
    if with_res:
        xn = xn + x_ref[...]
    return jnp.maximum(xn, 0.0)


def _mc_layer_body(segm_ref, cntm_ref, xm_ref, segc_ref, cntc_ref, xc_ref,
                   wlm_ref, blm_ref, wrm_ref, gm_ref, bm_ref,
                   wlc_ref, blc_ref, wrc_ref, gc_ref, bc_ref,
                   wzm_ref, bzm_ref, wzc_ref, bzc_ref,
                   xmo_ref, xco_ref, zmo_ref, zco_ref,
                   *, with_res, with_z):
    xm = _node_update(segm_ref, cntm_ref, xm_ref, wlm_ref, blm_ref, wrm_ref,
                      gm_ref, bm_ref, N_M, with_res)
    xc = _node_update(segc_ref, cntc_ref, xc_ref, wlc_ref, blc_ref, wrc_ref,
                      gc_ref, bc_ref, N_C, with_res)
    xmo_ref[...] = xm
    xco_ref[...] = xc
    if with_z:
        zmo_ref[...] = jnp.dot(xm, wzm_ref[...],
                               preferred_element_type=jnp.float32) + bzm_ref[...]
        zco_ref[...] = jnp.dot(xc, wzc_ref[...],
                               preferred_element_type=jnp.float32) + bzc_ref[...]
    else:
        zmo_ref[...] = jnp.zeros_like(zmo_ref)
        zco_ref[...] = jnp.zeros_like(zco_ref)


def _mc_layer(segm, cntm, xm, segc, cntc, xc, weights, with_res, with_z):
    body = functools.partial(_mc_layer_body, with_res=with_res, with_z=with_z)
    return pl.pallas_call(
        body,
        out_shape=[jax.ShapeDtypeStruct((N_MP, H), jnp.float32),
                   jax.ShapeDtypeStruct((N_CP, H), jnp.float32),
                   jax.ShapeDtypeStruct((N_MP, H), jnp.float32),
                   jax.ShapeDtypeStruct((N_CP, H), jnp.float32)],
    )(segm, cntm, xm, segc, cntc, xc, *weights)


def _lstm_dir(xs, wih_ref, whh_ref, b_ref):
    h = None
    c = None
    hs = []
    for t in range(4):
        gates = []
        for gi in range(4):
            acc = jnp.dot(xs[t], wih_ref[gi],
                          preferred_element_type=jnp.float32) + b_ref[gi]
            if h is not None:
                acc = acc + jnp.dot(h, whh_ref[gi],
                                    preferred_element_type=jnp.float32)
            gates.append(acc)
        gi_, gf_, gg_, go_ = gates
        if c is None:
            c = jax.nn.sigmoid(gi_) * jnp.tanh(gg_)
        else:
            c = jax.nn.sigmoid(gf_) * c + jax.nn.sigmoid(gi_) * jnp.tanh(gg_)
        h = jax.nn.sigmoid(go_) * jnp.tanh(c)
        hs.append(h)
    return hs


def _final_body(x0_ref, x1_ref, x2_ref, h2_ref, st2_ref, g2bn_ref, b2bn_ref,
                wihf_ref, whhf_ref, bf_ref, wihb_ref, whhb_ref, bb_ref,
                waf_ref, wab_ref, wpre_ref, bpre_ref, wcls_ref, bcls_ref,
                o_ref):
    x2 = x2_ref[...]
    xn2 = _bn_from_stats(h2_ref[...], st2_ref, g2bn_ref, b2bn_ref,
                         float(N_TX))
    x3 = jnp.maximum(xn2 + x2, 0.0)
    xs = [x0_ref[...], x1_ref[...], x2, x3]
    fw = _lstm_dir(xs, wihf_ref, whhf_ref, bf_ref)
    bwr = _lstm_dir(xs[::-1], wihb_ref, whhb_ref, bb_ref)
    bw = bwr[::-1]
    a = [jnp.sum(fw[t] * waf_ref[...], axis=1, keepdims=True)
         + jnp.sum(bw[t] * wab_ref[...], axis=1, keepdims=True)
         for t in range(4)]
    amax = jnp.maximum(jnp.maximum(a[0], a[1]), jnp.maximum(a[2], a[3]))
    e = [jnp.exp(av - amax) for av in a]
    denom = e[0] + e[1] + e[2] + e[3]
    xt = sum((e[t] / denom) * xs[t] for t in range(4))
    pre = jnp.maximum(
        jnp.dot(xt, wpre_ref[...], preferred_element_type=jnp.float32)
        + bpre_ref[...], 0.0)
    o_ref[...] = (jnp.dot(pre, wcls_ref[...],
                          preferred_element_type=jnp.float32) + bcls_ref[...])


def _final(xts, lw):
    full2 = lambda shape: pl.BlockSpec(shape, lambda i: (0, 0))
    full3 = lambda shape: pl.BlockSpec(shape, lambda i: (0, 0, 0))
    blk = pl.BlockSpec((BT, H), lambda i: (i, 0))
    return pl.pallas_call(
        _final_body,
        grid=(NBLK,),
        in_specs=[blk, blk, blk, blk,
                  full2((8, H)), full2((1, H)), full2((1, H)),
                  full3((4, H, LSTM_H)), full3((4, LSTM_H, LSTM_H)),
                  full3((4, 1, LSTM_H)),
                  full3((4, H, LSTM_H)), full3((4, LSTM_H, LSTM_H)),
                  full3((4, 1, LSTM_H)),
                  full2((1, LSTM_H)), full2((1, LSTM_H)),
                  full2((H, H)), full2((1, H)),
                  full2((H, N_C)), full2((1, N_C))],
        out_specs=pl.BlockSpec((BT, N_C), lambda i: (i, 0)),
        out_shape=jax.ShapeDtypeStruct((N_TX, N_C), jnp.float32),
    )(*xts, *lw)


def kernel(x_transaction, x_merchant, x_category, ei_tm, ei_tc, ei_mt, ei_ct,
           params):
    f32 = jnp.float32
    merch = ei_tm[1].astype(jnp.int32)
    cat = ei_tc[1].astype(jnp.int32)

    x_tp = jnp.pad(x_transaction, ((0, N_TXP - N_TX), (0, 0)))
    x_mp = jnp.pad(x_merchant, ((0, N_MP - N_M), (0, 0)))
    x_cp = jnp.pad(x_category, ((0, N_CP - N_C), (0, 0)))
    midx = jnp.pad(merch, (0, N_TXP - N_TX),
                   constant_values=N_M).reshape(NW, K, CH)
    cidx = jnp.pad(cat, (0, N_TXP - N_TX),
                   constant_values=N_C).reshape(NW, K, CH)
    zer_h = jnp.zeros((320, H), f32)
    ones_h = jnp.ones((CH, H), f32)

    p = params
    row = lambda v: v.reshape(1, -1)

    cntm, cntc = _sc_counts(midx, cidx, ones_h, zer_h)
    cntm_s = cntm[:, :N_MP, :]
    cntc_s = cntc[:, :N_CP, :]

    x_t = _enc_tx(x_tp, p['enc']['transaction']['W'],
                  row(p['enc']['transaction']['b']))
    cv0 = p['convs'][0]
    xm, xc, zm, zc = _mc_enc(
        x_mp, x_cp,
        p['enc']['merchant']['W'], row(p['enc']['merchant']['b']),
        p['enc']['category']['W'], row(p['enc']['category']['b']),
        cv0['mt']['Wl'], row(cv0['mt']['bl']),
        cv0['ct']['Wl'], row(cv0['ct']['bl']))

    xts = [x_t]
    for i in range(L):
        cv = p['convs'][i]
        bn = p['bn'][i]
        segm, segc, g = _sc_layer(x_t, zm, zc, midx, cidx, zer_h)
        wr_sum = cv['mt']['Wr'] + cv['ct']['Wr']
        h_pre, st = _tpre(x_t, g, wr_sum)
        if i == L - 1:
            h2, st2 = h_pre, st
            break
        x_t_new = _tapply(h_pre, st,
                          row(bn['transaction']['gamma']),
                          row(bn['transaction']['beta']),
                          res=x_t if i > 0 else None)
        cvn = p['convs'][i + 1]
        weights = [
            cv['tm']['Wl'], row(cv['tm']['bl']), cv['tm']['Wr'],
            row(bn['merchant']['gamma']), row(bn['merchant']['beta']),
            cv['tc']['Wl'], row(cv['tc']['bl']), cv['tc']['Wr'],
            row(bn['category']['gamma']), row(bn['category']['beta']),
            cvn['mt']['Wl'], row(cvn['mt']['bl']),
            cvn['ct']['Wl'], row(cvn['ct']['bl']),
        ]
        xm, xc, zm, zc = _mc_layer(segm[:, :N_MP, :], cntm_s, xm,
                                   segc[:, :N_CP, :], cntc_s, xc,
                                   weights, with_res=i > 0, with_z=True)
        x_t = x_t_new
        xts.append(x_t)

    def lstm_prep(lp):
        wih = lp['Wih'].reshape(4, LSTM_H, H).transpose(0, 2, 1)
        whh = lp['Whh'].reshape(4, LSTM_H, LSTM_H).transpose(0, 2, 1)
        b = (lp['bih'] + lp['bhh']).reshape(4, 1, LSTM_H)
        return wih, whh, b

    wihf, whhf, bf = lstm_prep(p['lstm']['fw'])
    wihb, whhb, bb = lstm_prep(p['lstm']['bw'])
    waf = p['att']['W'][:LSTM_H, 0].reshape(1, LSTM_H)
    wab = p['att']['W'][LSTM_H:, 0].reshape(1, LSTM_H)
    lw = [wihf, whhf, bf, wihb, whhb, bb, waf, wab,
          p['pre']['W'], row(p['pre']['b']),
          p['cls']['W'], row(p['cls']['b'])]
    bn2 = p['bn'][L - 1]['transaction']
    ins = xts + [h2, st2, row(bn2['gamma']), row(bn2['beta'])]
    return _final(ins, lw)


# speedup vs baseline: 5.1903x; 1.0740x over previous
import functools

import jax
import jax.numpy as jnp
from jax import lax
from jax.experimental import pallas as pl
from jax.experimental.pallas import tpu as pltpu
from jax.experimental.pallas import tpu_sc as plsc

N_TX, N_M, N_C = 50000, 5000, 400
D_IN, H, L = 64, 128, 3
LSTM_H = (L * H) // 2

NC, NS = 2, 16
NW = NC * NS
CH = 112
K = 14
ROWS_W = CH * K
N_TXP = NW * ROWS_W
N_MP = 5008
N_CP = 408
SEGM = 5120
SEGC = 512
BT = 512
NBLK = N_TXP // BT

@functools.cache
def _sc_mesh():
    return plsc.VectorSubcoreMesh(core_axis_name="c", subcore_axis_name="s",
                                  num_cores=NC, num_subcores=NS)


def _sc_counts_body(midx_h, cidx_h, ones_h, zer_h, cntm_o, cntc_o,
                    midx, cidx, bufone, segm_s, segc_s):
    c = lax.axis_index("c")
    s = lax.axis_index("s")
    wid = s * NC + c
    pltpu.sync_copy(zer_h, segm_s.at[pl.ds(s * 320, 320)])
    pltpu.sync_copy(zer_h.at[pl.ds(0, 32)], segc_s.at[pl.ds(s * 32, 32)])
    pltpu.sync_copy(ones_h, bufone)
    pltpu.sync_copy(midx_h.at[wid], midx)
    pltpu.sync_copy(cidx_h.at[wid], cidx)
    plsc.subcore_barrier()

    def chunk(j, carry):
        pltpu.sync_copy(bufone, segm_s.at[midx.at[j]], add=True)
        pltpu.sync_copy(bufone, segc_s.at[cidx.at[j]], add=True)
        return carry

    lax.fori_loop(0, K, chunk, 0)
    plsc.subcore_barrier()
    pltpu.sync_copy(segm_s.at[pl.ds(s * 320, 320)], cntm_o.at[c, pl.ds(s * 320, 320)])
    pltpu.sync_copy(segc_s.at[pl.ds(s * 32, 32)], cntc_o.at[c, pl.ds(s * 32, 32)])


def _sc_layer_body(xt_h, zm_h, zc_h, midx_h, cidx_h, zer_h,
                   segm_o, segc_o, g_o,
                   midx, cidx, bufx, bufa, bufb, segm_s, segc_s, *sems):
    c = lax.axis_index("c")
    s = lax.axis_index("s")
    wid = s * NC + c
    pltpu.sync_copy(zer_h, segm_s.at[pl.ds(s * 320, 320)])
    pltpu.sync_copy(zer_h.at[pl.ds(0, 32)], segc_s.at[pl.ds(s * 32, 32)])
    pltpu.sync_copy(midx_h.at[wid], midx)
    pltpu.sync_copy(cidx_h.at[wid], cidx)
    plsc.subcore_barrier()

    slx = sems[0:2]
    sga = sems[2:4]
    ssm = sems[4:6]
    ssc = sems[6:8]
    sgm = sems[8:10]
    sgb = sems[10]
    sgc = sems[11]

    def base(j):
        return wid * ROWS_W + j * CH

    h_lx, h_ga, h_gb, h_sm, h_sc, h_gm, h_gc = {}, {}, {}, {}, {}, {}, {}
    h_lx[0] = pltpu.async_copy(xt_h.at[pl.ds(base(0), CH)], bufx.at[0],
                               slx[0])
    h_ga[0] = pltpu.async_copy(zm_h.at[midx.at[0]], bufa.at[0], sga[0])
    h_gb[0] = pltpu.async_copy(zc_h.at[cidx.at[0]], bufb, sgb)
    for j in range(K):
        sl = j % 2
        h_ga[j].wait()
        h_gm[j] = pltpu.async_copy(bufa.at[sl], g_o.at[0, pl.ds(base(j), CH)],
                                   sgm[sl])
        h_gb[j].wait()
        h_gc[j] = pltpu.async_copy(bufb, g_o.at[1, pl.ds(base(j), CH)], sgc)
        h_lx[j].wait()
        h_sm[j] = pltpu.async_copy(bufx.at[sl], segm_s.at[midx.at[j]],
                                   ssm[sl], add=True)
        h_sc[j] = pltpu.async_copy(bufx.at[sl], segc_s.at[cidx.at[j]],
                                   ssc[sl], add=True)
        if j + 1 < K:
            osl = (j + 1) % 2
            if j >= 1:
                h_sm[j - 1].wait()
                h_sc[j - 1].wait()
            h_lx[j + 1] = pltpu.async_copy(xt_h.at[pl.ds(base(j + 1), CH)],
                                           bufx.at[osl], slx[osl])
            if j >= 1:
                h_gm[j - 1].wait()
            h_ga[j + 1] = pltpu.async_copy(zm_h.at[midx.at[j + 1]],
                                           bufa.at[osl], sga[osl])
            h_gc[j].wait()
            h_gb[j + 1] = pltpu.async_copy(zc_h.at[cidx.at[j + 1]], bufb, sgb)
    h_sm[K - 2].wait()
    h_sc[K - 2].wait()
    h_sm[K - 1].wait()
    h_sc[K - 1].wait()
    h_gm[K - 2].wait()
    h_gm[K - 1].wait()
    h_gc[K - 1].wait()
    plsc.subcore_barrier()
    pltpu.sync_copy(segm_s.at[pl.ds(s * 320, 320)], segm_o.at[c, pl.ds(s * 320, 320)])
    pltpu.sync_copy(segc_s.at[pl.ds(s * 32, 32)], segc_o.at[c, pl.ds(s * 32, 32)])


def _sc_counts(midx, cidx, ones_h, zer_h):
    f = pl.kernel(
        _sc_counts_body,
        out_type=[jax.ShapeDtypeStruct((NC, SEGM, H), jnp.float32),
                  jax.ShapeDtypeStruct((NC, SEGC, H), jnp.float32)],
        mesh=_sc_mesh(),
        scratch_types=[
            pltpu.VMEM((K, CH), jnp.int32),
            pltpu.VMEM((K, CH), jnp.int32),
            pltpu.VMEM((CH, H), jnp.float32),
            pltpu.VMEM_SHARED((SEGM, H), jnp.float32),
            pltpu.VMEM_SHARED((SEGC, H), jnp.float32),
        ],
    )
    return f(midx, cidx, ones_h, zer_h)


def _sc_layer(xt, zm, zc, midx, cidx, zer_h):
    f = pl.kernel(
        _sc_layer_body,
        out_type=[jax.ShapeDtypeStruct((NC, SEGM, H), jnp.float32),
                  jax.ShapeDtypeStruct((NC, SEGC, H), jnp.float32),
                  jax.ShapeDtypeStruct((2, N_TXP, H), jnp.float32)],
        mesh=_sc_mesh(),
        scratch_types=[
            pltpu.VMEM((K, CH), jnp.int32),
            pltpu.VMEM((K, CH), jnp.int32),
            pltpu.VMEM((2, CH, H), jnp.float32),
            pltpu.VMEM((2, CH, H), jnp.float32),
            pltpu.VMEM((CH, H), jnp.float32),
            pltpu.VMEM_SHARED((SEGM, H), jnp.float32),
            pltpu.VMEM_SHARED((SEGC, H), jnp.float32),
        ] + [pltpu.SemaphoreType.DMA] * 12,
    )
    return f(xt, zm, zc, midx, cidx, zer_h)


def _enc_body(x_ref, w_ref, b_ref, o_ref):
    o_ref[...] = jnp.maximum(
        jnp.dot(x_ref[...], w_ref[...], preferred_element_type=jnp.float32)
        + b_ref[...], 0.0)


def _enc_tx(x, w, b):
    return pl.pallas_call(
        _enc_body,
        grid=(NBLK,),
        in_specs=[pl.BlockSpec((BT, D_IN), lambda i: (i, 0)),
                  pl.BlockSpec((D_IN, H), lambda i: (0, 0)),
                  pl.BlockSpec((1, H), lambda i: (0, 0))],
        out_specs=pl.BlockSpec((BT, H), lambda i: (i, 0)),
        out_shape=jax.ShapeDtypeStruct((N_TXP, H), jnp.float32),
    )(x, w, b)


def _tpre_body(x_ref, g_ref, w_ref, h_ref, st_ref, acc_ref):
    i = pl.program_id(0)
    h = (jnp.dot(x_ref[...], w_ref[...], preferred_element_type=jnp.float32)
         + g_ref[0] + g_ref[1])
    h_ref[...] = h

    @pl.when(i == 0)
    def _():
        acc_ref[...] = jnp.zeros_like(acc_ref)

    ridx = i * BT + lax.broadcasted_iota(jnp.int32, (BT, 1), 0)
    m = (ridx < N_TX).astype(jnp.float32)
    hm = h * m
    acc_ref[0:1, :] += jnp.sum(hm, axis=0, keepdims=True)
    acc_ref[1:2, :] += jnp.sum(hm * h, axis=0, keepdims=True)

    @pl.when(i == NBLK - 1)
    def _():
        st_ref[...] = acc_ref[...]


def _tpre(x, g, w):
    return pl.pallas_call(
        _tpre_body,
        grid=(NBLK,),
        in_specs=[pl.BlockSpec((BT, H), lambda i: (i, 0)),
                  pl.BlockSpec((2, BT, H), lambda i: (0, i, 0)),
                  pl.BlockSpec((H, H), lambda i: (0, 0))],
        out_specs=[pl.BlockSpec((BT, H), lambda i: (i, 0)),
                   pl.BlockSpec((8, H), lambda i: (0, 0))],
        out_shape=[jax.ShapeDtypeStruct((N_TXP, H), jnp.float32),
                   jax.ShapeDtypeStruct((8, H), jnp.float32)],
        scratch_shapes=[pltpu.VMEM((8, H), jnp.float32)],
    )(x, g, w)


def _bn_from_stats(h, st_ref, gamma_ref, beta_ref, n):
    mu = st_ref[0:1, :] / n
    ex2 = st_ref[1:2, :] / n
    var = ex2 - mu * mu
    inv = lax.rsqrt(var + 1e-5)
    return (h - mu) * inv * gamma_ref[...] + beta_ref[...]


def _tapply_body_res(h_ref, st_ref, g_ref, b_ref, r_ref, o_ref):
    xn = _bn_from_stats(h_ref[...], st_ref, g_ref, b_ref, float(N_TX))
    o_ref[...] = jnp.maximum(xn + r_ref[...], 0.0)


def _tapply_body(h_ref, st_ref, g_ref, b_ref, o_ref):
    xn = _bn_from_stats(h_ref[...], st_ref, g_ref, b_ref, float(N_TX))
    o_ref[...] = jnp.maximum(xn, 0.0)


def _tapply(h, st, gamma, beta, res=None):
    specs = [pl.BlockSpec((BT, H), lambda i: (i, 0)),
             pl.BlockSpec((8, H), lambda i: (0, 0)),
             pl.BlockSpec((1, H), lambda i: (0, 0)),
             pl.BlockSpec((1, H), lambda i: (0, 0))]
    args = [h, st, gamma, beta]
    body = _tapply_body
    if res is not None:
        specs.append(pl.BlockSpec((BT, H), lambda i: (i, 0)))
        args.append(res)
        body = _tapply_body_res
    return pl.pallas_call(
        body,
        grid=(NBLK,),
        in_specs=specs,
        out_specs=pl.BlockSpec((BT, H), lambda i: (i, 0)),
        out_shape=jax.ShapeDtypeStruct((N_TXP, H), jnp.float32),
    )(*args)


def _mc_enc_body(xm_ref, xc_ref, wm_ref, bm_ref, wc_ref, bc_ref,
                 wzm_ref, bzm_ref, wzc_ref, bzc_ref,
                 xmo_ref, xco_ref, zmo_ref, zco_ref):
    xm = jnp.maximum(
        jnp.dot(xm_ref[...], wm_ref[...], preferred_element_type=jnp.float32)
        + bm_ref[...], 0.0)
    xc = jnp.maximum(
        jnp.dot(xc_ref[...], wc_ref[...], preferred_element_type=jnp.float32)
        + bc_ref[...], 0.0)
    xmo_ref[...] = xm
    xco_ref[...] = xc
    zmo_ref[...] = jnp.dot(xm, wzm_ref[...],
                           preferred_element_type=jnp.float32) + bzm_ref[...]
    zco_ref[...] = jnp.dot(xc, wzc_ref[...],
                           preferred_element_type=jnp.float32) + bzc_ref[...]


def _mc_enc(xm, xc, wm, bm, wc, bc, wzm, bzm, wzc, bzc):
    return pl.pallas_call(
        _mc_enc_body,
        out_shape=[jax.ShapeDtypeStruct((N_MP, H), jnp.float32),
                   jax.ShapeDtypeStruct((N_CP, H), jnp.float32),
                   jax.ShapeDtypeStruct((N_MP, H), jnp.float32),
                   jax.ShapeDtypeStruct((N_CP, H), jnp.float32)],
    )(xm, xc, wm, bm, wc, bc, wzm, bzm, wzc, bzc)


def _node_update(seg_ref, cnt_ref, x_ref, wl_ref, bl_ref, wr_ref,
                 gam_ref, bet_ref, n_valid, with_res):
    seg = seg_ref[0] + seg_ref[1]
    cnt = jnp.maximum(cnt_ref[0] + cnt_ref[1], 1.0)
    agg = seg / cnt
    h = (jnp.dot(agg, wl_ref[...], preferred_element_type=jnp.float32)
         + bl_ref[...]
         + jnp.dot(x_ref[...], wr_ref[...], preferred_element_type=jnp.float32))
    rows = h.shape[0]
    ridx = lax.broadcasted_iota(jnp.int32, (rows, 1), 0)
    m = (ridx < n_valid).astype(jnp.float32)
    hm = h * m
    n = float(n_valid)
    mu = jnp.sum(hm, axis=0, keepdims=True) / n
    ex2 = jnp.sum(hm * h, axis=0, keepdims=True) / n
    var = ex2 - mu * mu
    xn = (h - mu) * lax.rsqrt(var + 1e-5) * gam_ref[...] + bet_ref[...]
    if with_res:
        xn = xn + x_ref[...]
    return jnp.maximum(xn, 0.0)


def _mc_layer_body(segm_ref, cntm_ref, xm_ref, segc_ref, cntc_ref, xc_ref,
                   wlm_ref, blm_ref, wrm_ref, gm_ref, bm_ref,
                   wlc_ref, blc_ref, wrc_ref, gc_ref, bc_ref,
                   wzm_ref, bzm_ref, wzc_ref, bzc_ref,
                   xmo_ref, xco_ref, zmo_ref, zco_ref,
                   *, with_res, with_z):
    xm = _node_update(segm_ref, cntm_ref, xm_ref, wlm_ref, blm_ref, wrm_ref,
                      gm_ref, bm_ref, N_M, with_res)
    xc = _node_update(segc_ref, cntc_ref, xc_ref, wlc_ref, blc_ref, wrc_ref,
                      gc_ref, bc_ref, N_C, with_res)
    xmo_ref[...] = xm
    xco_ref[...] = xc
    if with_z:
        zmo_ref[...] = jnp.dot(xm, wzm_ref[...],
                               preferred_element_type=jnp.float32) + bzm_ref[...]
        zco_ref[...] = jnp.dot(xc, wzc_ref[...],
                               preferred_element_type=jnp.float32) + bzc_ref[...]
    else:
        zmo_ref[...] = jnp.zeros_like(zmo_ref)
        zco_ref[...] = jnp.zeros_like(zco_ref)


def _mc_layer(segm, cntm, xm, segc, cntc, xc, weights, with_res, with_z):
    body = functools.partial(_mc_layer_body, with_res=with_res, with_z=with_z)
    return pl.pallas_call(
        body,
        out_shape=[jax.ShapeDtypeStruct((N_MP, H), jnp.float32),
                   jax.ShapeDtypeStruct((N_CP, H), jnp.float32),
                   jax.ShapeDtypeStruct((N_MP, H), jnp.float32),
                   jax.ShapeDtypeStruct((N_CP, H), jnp.float32)],
    )(segm, cntm, xm, segc, cntc, xc, *weights)


def _lstm_dir(xs, wih_ref, whh_ref, b_ref):
    h = None
    c = None
    hs = []
    for t in range(4):
        gates = []
        for gi in range(4):
            acc = jnp.dot(xs[t], wih_ref[gi],
                          preferred_element_type=jnp.float32) + b_ref[gi]
            if h is not None:
                acc = acc + jnp.dot(h, whh_ref[gi],
                                    preferred_element_type=jnp.float32)
            gates.append(acc)
        gi_, gf_, gg_, go_ = gates
        if c is None:
            c = jax.nn.sigmoid(gi_) * jnp.tanh(gg_)
        else:
            c = jax.nn.sigmoid(gf_) * c + jax.nn.sigmoid(gi_) * jnp.tanh(gg_)
        h = jax.nn.sigmoid(go_) * jnp.tanh(c)
        hs.append(h)
    return hs


def _final_body(x0_ref, x1_ref, x2_ref, h2_ref, st2_ref, g2bn_ref, b2bn_ref,
                wihf_ref, whhf_ref, bf_ref, wihb_ref, whhb_ref, bb_ref,
                waf_ref, wab_ref, wpre_ref, bpre_ref, wcls_ref, bcls_ref,
                o_ref):
    x2 = x2_ref[...]
    xn2 = _bn_from_stats(h2_ref[...], st2_ref, g2bn_ref, b2bn_ref,
                         float(N_TX))
    x3 = jnp.maximum(xn2 + x2, 0.0)
    xs = [x0_ref[...], x1_ref[...], x2, x3]
    fw = _lstm_dir(xs, wihf_ref, whhf_ref, bf_ref)
    bwr = _lstm_dir(xs[::-1], wihb_ref, whhb_ref, bb_ref)
    bw = bwr[::-1]
    a = [jnp.sum(fw[t] * waf_ref[...], axis=1, keepdims=True)
         + jnp.sum(bw[t] * wab_ref[...], axis=1, keepdims=True)
         for t in range(4)]
    amax = jnp.maximum(jnp.maximum(a[0], a[1]), jnp.maximum(a[2], a[3]))
    e = [jnp.exp(av - amax) for av in a]
    denom = e[0] + e[1] + e[2] + e[3]
    xt = sum((e[t] / denom) * xs[t] for t in range(4))
    pre = jnp.maximum(
        jnp.dot(xt, wpre_ref[...], preferred_element_type=jnp.float32)
        + bpre_ref[...], 0.0)
    o_ref[...] = (jnp.dot(pre, wcls_ref[...],
                          preferred_element_type=jnp.float32) + bcls_ref[...])


def _final(xts, lw):
    full2 = lambda shape: pl.BlockSpec(shape, lambda i: (0, 0))
    full3 = lambda shape: pl.BlockSpec(shape, lambda i: (0, 0, 0))
    blk = pl.BlockSpec((BT, H), lambda i: (i, 0))
    return pl.pallas_call(
        _final_body,
        grid=(NBLK,),
        in_specs=[blk, blk, blk, blk,
                  full2((8, H)), full2((1, H)), full2((1, H)),
                  full3((4, H, LSTM_H)), full3((4, LSTM_H, LSTM_H)),
                  full3((4, 1, LSTM_H)),
                  full3((4, H, LSTM_H)), full3((4, LSTM_H, LSTM_H)),
                  full3((4, 1, LSTM_H)),
                  full2((1, LSTM_H)), full2((1, LSTM_H)),
                  full2((H, H)), full2((1, H)),
                  full2((H, N_C)), full2((1, N_C))],
        out_specs=pl.BlockSpec((BT, N_C), lambda i: (i, 0)),
        out_shape=jax.ShapeDtypeStruct((N_TX, N_C), jnp.float32),
    )(*xts, *lw)


def kernel(x_transaction, x_merchant, x_category, ei_tm, ei_tc, ei_mt, ei_ct,
           params):
    f32 = jnp.float32
    merch = ei_tm[1].astype(jnp.int32)
    cat = ei_tc[1].astype(jnp.int32)

    x_tp = jnp.pad(x_transaction, ((0, N_TXP - N_TX), (0, 0)))
    x_mp = jnp.pad(x_merchant, ((0, N_MP - N_M), (0, 0)))
    x_cp = jnp.pad(x_category, ((0, N_CP - N_C), (0, 0)))
    midx = jnp.pad(merch, (0, N_TXP - N_TX),
                   constant_values=N_M).reshape(NW, K, CH)
    cidx = jnp.pad(cat, (0, N_TXP - N_TX),
                   constant_values=N_C).reshape(NW, K, CH)
    zer_h = jnp.zeros((320, H), f32)
    ones_h = jnp.ones((CH, H), f32)

    p = params
    row = lambda v: v.reshape(1, -1)

    cntm, cntc = _sc_counts(midx, cidx, ones_h, zer_h)
    cntm_s = cntm[:, :N_MP, :]
    cntc_s = cntc[:, :N_CP, :]

    x_t = _enc_tx(x_tp, p['enc']['transaction']['W'],
                  row(p['enc']['transaction']['b']))
    cv0 = p['convs'][0]
    xm, xc, zm, zc = _mc_enc(
        x_mp, x_cp,
        p['enc']['merchant']['W'], row(p['enc']['merchant']['b']),
        p['enc']['category']['W'], row(p['enc']['category']['b']),
        cv0['mt']['Wl'], row(cv0['mt']['bl']),
        cv0['ct']['Wl'], row(cv0['ct']['bl']))

    xts = [x_t]
    for i in range(L):
        cv = p['convs'][i]
        bn = p['bn'][i]
        segm, segc, g = _sc_layer(x_t, zm, zc, midx, cidx, zer_h)
        wr_sum = cv['mt']['Wr'] + cv['ct']['Wr']
        h_pre, st = _tpre(x_t, g, wr_sum)
        if i == L - 1:
            h2, st2 = h_pre, st
            break
        x_t_new = _tapply(h_pre, st,
                          row(bn['transaction']['gamma']),
                          row(bn['transaction']['beta']),
                          res=x_t if i > 0 else None)
        cvn = p['convs'][i + 1]
        weights = [
            cv['tm']['Wl'], row(cv['tm']['bl']), cv['tm']['Wr'],
            row(bn['merchant']['gamma']), row(bn['merchant']['beta']),
            cv['tc']['Wl'], row(cv['tc']['bl']), cv['tc']['Wr'],
            row(bn['category']['gamma']), row(bn['category']['beta']),
            cvn['mt']['Wl'], row(cvn['mt']['bl']),
            cvn['ct']['Wl'], row(cvn['ct']['bl']),
        ]
        xm, xc, zm, zc = _mc_layer(segm[:, :N_MP, :], cntm_s, xm,
                                   segc[:, :N_CP, :], cntc_s, xc,
                                   weights, with_res=i > 0, with_z=True)
        x_t = x_t_new
        xts.append(x_t)

    def lstm_prep(lp):
        wih = lp['Wih'].reshape(4, LSTM_H, H).transpose(0, 2, 1)
        whh = lp['Whh'].reshape(4, LSTM_H, LSTM_H).transpose(0, 2, 1)
        b = (lp['bih'] + lp['bhh']).reshape(4, 1, LSTM_H)
        return wih, whh, b

    wihf, whhf, bf = lstm_prep(p['lstm']['fw'])
    wihb, whhb, bb = lstm_prep(p['lstm']['bw'])
    waf = p['att']['W'][:LSTM_H, 0].reshape(1, LSTM_H)
    wab = p['att']['W'][LSTM_H:, 0].reshape(1, LSTM_H)
    lw = [wihf, whhf, bf, wihb, whhb, bb, waf, wab,
          p['pre']['W'], row(p['pre']['b']),
          p['cls']['W'], row(p['cls']['b'])]
    bn2 = p['bn'][L - 1]['transaction']
    ins = xts + [h2, st2, row(bn2['gamma']), row(bn2['beta'])]
    return _final(ins, lw)
